# Initial kernel scaffold; baseline (speedup 1.0000x reference)
import jax
import jax.numpy as jnp
from jax.experimental import pallas as pl


def kernel(predicted, target, mask):
    raise NotImplementedError("write your pallas kernel here")


# speedup vs baseline: 8.0000x; 8.0000x over previous
import functools

import jax
import jax.numpy as jnp
from jax import lax
from jax.experimental import pallas as pl
from jax.experimental.pallas import tpu as pltpu
from jax.experimental.pallas import tpu_sc as plsc

B = 64
N = 512 * 512
K = 1 + round(0.01 * 99.0 * (N - 1))

NC = 2
NS = 16
NW = NC * NS
SPT = B // NW

NB = 4096
NB_C = 128
L = 16
CH = 4096
NCH = N // CH
VPC = CH // L


def _zero_hist(hist):
    def body(i, _):
        hist[pl.ds(i * L, L)] = jnp.zeros((L,), jnp.int32)
        return 0
    lax.fori_loop(0, NB * L // L, body, 0)


def _select(hist, r, nb):
    zero = jnp.int32(0)

    def body(j, carry):
        cum, bstar, cumbef = carry
        v = jnp.zeros((L,), jnp.int32)
        for bank in range(L):
            v = v + hist[pl.ds(bank * NB + j * L, L)]
        cv = plsc.cumsum(v) + cum
        mlt = cv < r
        ones_v = jnp.ones((L,), jnp.int32)
        zeros_v = jnp.zeros((L,), jnp.int32)
        bstar = bstar + jnp.sum(jnp.where(mlt, ones_v, zeros_v))
        cumbef = cumbef + jnp.sum(jnp.where(mlt, v, zeros_v))
        cum = cum + jnp.sum(v)
        return (cum, bstar, cumbef)

    cum, bstar, cumbef = lax.fori_loop(0, nb // L, body, (zero, zero, zero))
    return bstar, r - cumbef


def _sc_body(pred_hbm, tgt_hbm, mask_hbm, qbits_hbm, sums_hbm, loss_hbm,
             bufp, buft, bufm, bufl, hist, outbuf_i, outbuf_f):
    wid = lax.axis_index("s") * NC + lax.axis_index("c")
    lane = lax.broadcasted_iota(jnp.int32, (L,), 0)
    ones_i = jnp.ones((L,), jnp.int32)

    results = []
    for local in range(SPT):
        s = wid * SPT + local

        _zero_hist(hist)

        def chunk_a(c, acc):
            off = pl.multiple_of(c * CH, CH)
            pltpu.sync_copy(pred_hbm.at[s, pl.ds(off, CH)], bufp)
            pltpu.sync_copy(tgt_hbm.at[s, pl.ds(off, CH)], buft)
            pltpu.sync_copy(mask_hbm.at[s, pl.ds(off, CH)], bufm)

            def vbody(i, acc):
                p = bufp[pl.ds(i * L, L)]
                t = buft[pl.ds(i * L, L)]
                m = bufm[pl.ds(i * L, L)]
                lv = m * lax.abs(p - t)
                bufl[pl.ds(i * L, L)] = lv
                bits = lax.bitcast_convert_type(lv, jnp.int32)
                d = lax.shift_right_logical(bits, 19)
                plsc.addupdate_scatter(hist, [lane * NB + d], ones_i)
                return acc + lv

            acc = lax.fori_loop(0, VPC, vbody, acc)
            pltpu.sync_copy(bufl, loss_hbm.at[s, pl.ds(off, CH)])
            return acc

        acc = lax.fori_loop(0, NCH, chunk_a, jnp.zeros((L,), jnp.float32))
        ssum = jnp.sum(acc)
        b1, r2 = _select(hist, jnp.int32(K), NB)

        _zero_hist(hist)

        def chunk_b(c, _):
            off = pl.multiple_of(c * CH, CH)
            pltpu.sync_copy(loss_hbm.at[s, pl.ds(off, CH)], bufl)

            def vbody(i, _):
                bits = lax.bitcast_convert_type(bufl[pl.ds(i * L, L)], jnp.int32)
                match = lax.shift_right_logical(bits, 19) == b1
                d = lax.bitwise_and(lax.shift_right_logical(bits, 7),
                                    jnp.int32(0xFFF))
                plsc.addupdate_scatter(hist, [lane * NB + d], ones_i,
                                       mask=match)
                return 0

            lax.fori_loop(0, VPC, vbody, 0)
            return 0

        lax.fori_loop(0, NCH, chunk_b, 0)
        b2, r3 = _select(hist, r2, NB)
        prefix24 = b1 * 4096 + b2

        _zero_hist(hist)

        def chunk_c(c, _):
            off = pl.multiple_of(c * CH, CH)
            pltpu.sync_copy(loss_hbm.at[s, pl.ds(off, CH)], bufl)

            def vbody(i, _):
                bits = lax.bitcast_convert_type(bufl[pl.ds(i * L, L)], jnp.int32)
                match = lax.shift_right_logical(bits, 7) == prefix24
                d = lax.bitwise_and(bits, jnp.int32(0x7F))
                plsc.addupdate_scatter(hist, [lane * NB + d], ones_i,
                                       mask=match)
                return 0

            lax.fori_loop(0, VPC, vbody, 0)
            return 0

        lax.fori_loop(0, NCH, chunk_c, 0)
        b3, _ = _select(hist, r3, NB_C)
        qbits = prefix24 * 128 + b3
        results.append((qbits, ssum))

    (q0, s0), (q1, s1) = results
    row_i = jnp.where(lane == 0, jnp.full((L,), q0, jnp.int32),
                      jnp.where(lane == 1, jnp.full((L,), q1, jnp.int32),
                                jnp.zeros((L,), jnp.int32)))
    row_f = jnp.where(lane == 0, jnp.full((L,), s0),
                      jnp.where(lane == 1, jnp.full((L,), s1),
                                jnp.zeros((L,))))
    outbuf_i[...] = row_i
    outbuf_f[...] = row_f
    pltpu.sync_copy(outbuf_i, qbits_hbm.at[wid])
    pltpu.sync_copy(outbuf_f, sums_hbm.at[wid])


@jax.jit
def kernel(predicted, target, mask):
    pred2 = predicted.reshape(B, N)
    tgt2 = target.reshape(B, N)
    mask2 = mask.reshape(B, N)

    mesh = plsc.VectorSubcoreMesh(core_axis_name="c", subcore_axis_name="s",
                                  num_cores=NC, num_subcores=NS)
    qbits, sums, _loss_scratch = pl.kernel(
        _sc_body,
        out_type=[
            jax.ShapeDtypeStruct((NW, L), jnp.int32),
            jax.ShapeDtypeStruct((NW, L), jnp.float32),
            jax.ShapeDtypeStruct((B, N), jnp.float32),
        ],
        mesh=mesh,
        compiler_params=pltpu.CompilerParams(needs_layout_passes=False),
        scratch_types=[
            pltpu.VMEM((CH,), jnp.float32),
            pltpu.VMEM((CH,), jnp.float32),
            pltpu.VMEM((CH,), jnp.float32),
            pltpu.VMEM((CH,), jnp.float32),
            pltpu.VMEM((NB * L,), jnp.int32),
            pltpu.VMEM((L,), jnp.int32),
            pltpu.VMEM((L,), jnp.float32),
        ],
    )(pred2, tgt2, mask2)

    q_loss = lax.bitcast_convert_type(qbits[:, :SPT].reshape(B), jnp.float32)
    wmae = jnp.sum(sums[:, :SPT]) / (B * N)
    return (q_loss, wmae)


# speedup vs baseline: 19.2256x; 2.4032x over previous
import jax
import jax.numpy as jnp
from jax import lax
from jax.experimental import pallas as pl
from jax.experimental.pallas import tpu as pltpu
from jax.experimental.pallas import tpu_sc as plsc

B = 64
H = 512
W = 512
N = H * W
K = 1 + round(0.01 * 99.0 * (N - 1))

NC = 2
NS = 16
NW = NC * NS
SPT = B // NW

NB = 4096
NB_C = 128
L = 16
CH = 8192
NCH = N // CH
VPC = CH // L
CAP = 32768


def _tc_body(pred_ref, tgt_ref, mask_ref, loss_ref, sums_ref):
    lv = mask_ref[...] * lax.abs(pred_ref[...] - tgt_ref[...])
    loss_ref[...] = lv
    sums_ref[...] = jnp.full((1, 1, 128), jnp.sum(lv), jnp.float32)


def _tc_loss(pred, tgt, mask):
    return pl.pallas_call(
        _tc_body,
        grid=(B,),
        in_specs=[
            pl.BlockSpec((1, H, W), lambda b: (b, 0, 0)),
            pl.BlockSpec((1, H, W), lambda b: (b, 0, 0)),
            pl.BlockSpec((1, H, W), lambda b: (b, 0, 0)),
        ],
        out_specs=[
            pl.BlockSpec((1, H, W), lambda b: (b, 0, 0)),
            pl.BlockSpec((1, 1, 128), lambda b: (b, 0, 0)),
        ],
        out_shape=[
            jax.ShapeDtypeStruct((B, H, W), jnp.float32),
            jax.ShapeDtypeStruct((B, 1, 128), jnp.float32),
        ],
    )(pred, tgt, mask)


def _zero_hist(hist, nb):
    def body(i, _):
        for bank in range(L):
            hist[pl.ds(bank * NB + i * L, L)] = jnp.zeros((L,), jnp.int32)
        return 0
    lax.fori_loop(0, nb // L, body, 0)


def _select(hist, r, nb):
    zero = jnp.int32(0)

    def body(j, carry):
        cum, bstar, cumbef, cnt = carry
        v = jnp.zeros((L,), jnp.int32)
        for bank in range(L):
            v = v + hist[pl.ds(bank * NB + j * L, L)]
        cv = plsc.cumsum(v) + cum
        mlt = cv < r
        msel = jnp.logical_and(cv >= r, (cv - v) < r)
        ones_v = jnp.ones((L,), jnp.int32)
        zeros_v = jnp.zeros((L,), jnp.int32)
        bstar = bstar + jnp.sum(jnp.where(mlt, ones_v, zeros_v))
        cumbef = cumbef + jnp.sum(jnp.where(mlt, v, zeros_v))
        cnt = cnt + jnp.sum(jnp.where(msel, v, zeros_v))
        cum = cum + jnp.sum(v)
        return (cum, bstar, cumbef, cnt)

    cum, bstar, cumbef, cnt = lax.fori_loop(
        0, nb // L, body, (zero, zero, zero, zero))
    return bstar, r - cumbef, cnt


def _stream_pass(src_hbm, s, buf0, buf1, sem0, sem1, chunk_fn, init_carry):
    def start(c, buf, sem):
        off = pl.multiple_of(c * CH, CH)
        pltpu.async_copy(src_hbm.at[s, pl.ds(off, CH)], buf, sem)

    def wait(c, buf, sem):
        off = pl.multiple_of(c * CH, CH)
        pltpu.make_async_copy(src_hbm.at[s, pl.ds(off, CH)], buf, sem).wait()

    start(0, buf0, sem0)
    start(1, buf1, sem1)

    def body(c2, carry):
        c0 = c2 * 2
        wait(c0, buf0, sem0)
        carry = chunk_fn(buf0, c0, carry)

        @pl.when(c0 + 2 < NCH)
        def _():
            start(c0 + 2, buf0, sem0)

        wait(c0 + 1, buf1, sem1)
        carry = chunk_fn(buf1, c0 + 1, carry)

        @pl.when(c0 + 3 < NCH)
        def _():
            start(c0 + 3, buf1, sem1)

        return carry

    return lax.fori_loop(0, NCH // 2, body, init_carry)


def _sc_body(loss_hbm, qbits_hbm,
             buf0, buf1, cand, hist, outbuf_i, sem0, sem1):
    wid = lax.axis_index("s") * NC + lax.axis_index("c")
    lane = lax.broadcasted_iota(jnp.int32, (L,), 0)
    ones_i = jnp.ones((L,), jnp.int32)

    results = []
    for local in range(SPT):
        s = wid * SPT + local

        _zero_hist(hist, NB)

        def chunk_a(buf, c, carry):
            def vbody(i, _):
                bits = lax.bitcast_convert_type(buf[pl.ds(i * L, L)],
                                                jnp.int32)
                d = lax.shift_right_logical(bits, 19)
                plsc.addupdate_scatter(hist, [lane * NB + d], ones_i)
                return 0
            lax.fori_loop(0, VPC, vbody, 0)
            return carry

        _stream_pass(loss_hbm, s, buf0, buf1, sem0, sem1, chunk_a, 0)
        b1, r2, cnt1 = _select(hist, jnp.int32(K), NB)
        docap = cnt1 <= CAP

        _zero_hist(hist, NB)

        def chunk_b(buf, c, cnt):
            def vbody(i, cnt):
                bits = lax.bitcast_convert_type(buf[pl.ds(i * L, L)],
                                                jnp.int32)
                match = lax.shift_right_logical(bits, 19) == b1
                d = lax.bitwise_and(lax.shift_right_logical(bits, 7),
                                    jnp.int32(0xFFF))
                plsc.addupdate_scatter(hist, [lane * NB + d], ones_i,
                                       mask=match)

                @pl.when(docap)
                def _():
                    plsc.store_compressed(cand.at[pl.ds(cnt, L)], bits,
                                          mask=match)

                return cnt + jnp.sum(jnp.where(match, ones_i,
                                               jnp.zeros((L,), jnp.int32)))
            return lax.fori_loop(0, VPC, vbody, cnt)

        _stream_pass(loss_hbm, s, buf0, buf1, sem0, sem1, chunk_b,
                     jnp.int32(0))
        b2, r3, _cnt2 = _select(hist, r2, NB)
        prefix24 = b1 * 4096 + b2

        _zero_hist(hist, NB_C)

        @pl.when(docap)
        def _():
            nv = (cnt1 + (L - 1)) // L

            def vbody(i, _):
                bits = cand[pl.ds(i * L, L)]
                inb = (i * L + lane) < cnt1
                match = jnp.logical_and(
                    lax.shift_right_logical(bits, 7) == prefix24, inb)
                d = lax.bitwise_and(bits, jnp.int32(0x7F))
                plsc.addupdate_scatter(hist, [lane * NB + d], ones_i,
                                       mask=match)
                return 0

            lax.fori_loop(0, nv, vbody, 0)

        @pl.when(jnp.logical_not(docap))
        def _():
            def chunk_c(buf, c, carry):
                def vbody(i, _):
                    bits = lax.bitcast_convert_type(buf[pl.ds(i * L, L)],
                                                    jnp.int32)
                    match = lax.shift_right_logical(bits, 7) == prefix24
                    d = lax.bitwise_and(bits, jnp.int32(0x7F))
                    plsc.addupdate_scatter(hist, [lane * NB + d], ones_i,
                                           mask=match)
                    return 0
                lax.fori_loop(0, VPC, vbody, 0)
                return carry

            _stream_pass(loss_hbm, s, buf0, buf1, sem0, sem1, chunk_c, 0)

        b3, _r4, _c4 = _select(hist, r3, NB_C)
        results.append(prefix24 * 128 + b3)

    q0, q1 = results
    row_i = jnp.where(lane == 0, jnp.full((L,), q0, jnp.int32),
                      jnp.where(lane == 1, jnp.full((L,), q1, jnp.int32),
                                jnp.zeros((L,), jnp.int32)))
    outbuf_i[...] = row_i
    pltpu.sync_copy(outbuf_i, qbits_hbm.at[wid])


@jax.jit
def kernel(predicted, target, mask):
    pred3 = predicted.reshape(B, H, W)
    tgt3 = target.reshape(B, H, W)
    mask3 = mask.reshape(B, H, W)

    loss, sums = _tc_loss(pred3, tgt3, mask3)

    mesh = plsc.VectorSubcoreMesh(core_axis_name="c", subcore_axis_name="s",
                                  num_cores=NC, num_subcores=NS)
    qbits = pl.kernel(
        _sc_body,
        out_type=jax.ShapeDtypeStruct((NW, L), jnp.int32),
        mesh=mesh,
        compiler_params=pltpu.CompilerParams(needs_layout_passes=False),
        scratch_types=[
            pltpu.VMEM((CH,), jnp.float32),
            pltpu.VMEM((CH,), jnp.float32),
            pltpu.VMEM((CAP + L,), jnp.int32),
            pltpu.VMEM((NB * L,), jnp.int32),
            pltpu.VMEM((L,), jnp.int32),
            pltpu.SemaphoreType.DMA,
            pltpu.SemaphoreType.DMA,
        ],
    )(loss.reshape(B, N))

    q_loss = lax.bitcast_convert_type(qbits[:, :SPT].reshape(B), jnp.float32)
    wmae = jnp.sum(sums[:, 0, 0]) / (B * N)
    return (q_loss, wmae)
